# baseline (speedup 1.0000x reference)
import functools

import jax
import jax.numpy as jnp
from jax import lax
from jax.experimental import pallas as pl
from jax.experimental.pallas import tpu as pltpu
from jax.experimental.pallas import tpu_sc as plsc


def _dot_t(a, w):
    return lax.dot_general(a, w, (((1,), (1,)), ((), ())),
                           preferred_element_type=jnp.float32)


def _sc_gather_pair(emb_a, idx_a, emb_b, idx_b):
    n = idx_a.shape[0]
    h = emb_a.shape[1]
    info = plsc.get_sparse_core_info()
    nw = info.num_cores * info.num_subcores
    n_per = n // nw

    nc = n_per // 2

    @functools.partial(
        pl.kernel,
        out_type=(jax.ShapeDtypeStruct((n, h), jnp.float32),
                  jax.ShapeDtypeStruct((n, h), jnp.float32)),
        mesh=plsc.VectorSubcoreMesh(core_axis_name="c", subcore_axis_name="s"),
        scratch_types=[
            pltpu.VMEM((nc,), jnp.int32),
            pltpu.VMEM((nc,), jnp.int32),
            pltpu.VMEM((nc,), jnp.int32),
            pltpu.VMEM((nc,), jnp.int32),
            pltpu.VMEM((nc, h), jnp.float32),
            pltpu.VMEM((nc, h), jnp.float32),
            pltpu.SemaphoreType.DMA,
            pltpu.SemaphoreType.DMA,
        ],
    )
    def k(emb_a_hbm, idx_a_hbm, emb_b_hbm, idx_b_hbm, out_a, out_b,
          ia0, ia1, ib0, ib1, r0, r1, s0, s1):
        wid = lax.axis_index("s") * info.num_cores + lax.axis_index("c")
        base = wid * n_per
        pltpu.sync_copy(idx_a_hbm.at[pl.ds(base, nc)], ia0)
        pltpu.sync_copy(idx_a_hbm.at[pl.ds(base + nc, nc)], ia1)
        pltpu.sync_copy(idx_b_hbm.at[pl.ds(base, nc)], ib0)
        pltpu.sync_copy(idx_b_hbm.at[pl.ds(base + nc, nc)], ib1)
        c0 = pltpu.async_copy(emb_a_hbm.at[ia0], r0, s0)
        c1 = pltpu.async_copy(emb_a_hbm.at[ia1], r1, s1)
        c0.wait()
        pltpu.sync_copy(r0, out_a.at[pl.ds(base, nc)])
        c2 = pltpu.async_copy(emb_b_hbm.at[ib0], r0, s0)
        c1.wait()
        pltpu.sync_copy(r1, out_a.at[pl.ds(base + nc, nc)])
        c3 = pltpu.async_copy(emb_b_hbm.at[ib1], r1, s1)
        c2.wait()
        pltpu.sync_copy(r0, out_b.at[pl.ds(base, nc)])
        c3.wait()
        pltpu.sync_copy(r1, out_b.at[pl.ds(base + nc, nc)])

    return k(emb_a, idx_a, emb_b, idx_b)


def _split_gates(gates, hh):
    i = jax.nn.sigmoid(gates[:, :hh])
    f = jax.nn.sigmoid(gates[:, hh:2 * hh])
    g = jnp.tanh(gates[:, 2 * hh:3 * hh])
    o = jax.nn.sigmoid(gates[:, 3 * hh:])
    return i, f, g, o


def _masked_xw(x, ids3, w, bias):
    m, h = x.shape
    n4 = w.shape[0]
    bm, bn = 512, 2048

    def body(x_ref, ids_ref, w_ref, b_ref, o_ref):
        mask = (ids_ref[0, 0, :] != 0).astype(jnp.float32)
        o_ref[...] = _dot_t(x_ref[...] * mask[:, None], w_ref[...]) + b_ref[...]

    return pl.pallas_call(
        body,
        grid=(n4 // bn, m // bm),
        in_specs=[
            pl.BlockSpec((bm, h), lambda n, mm: (mm, 0)),
            pl.BlockSpec((1, 1, bm), lambda n, mm: (mm, 0, 0)),
            pl.BlockSpec((bn, h), lambda n, mm: (n, 0)),
            pl.BlockSpec((1, bn), lambda n, mm: (0, n)),
        ],
        out_specs=pl.BlockSpec((bm, bn), lambda n, mm: (mm, n)),
        out_shape=jax.ShapeDtypeStruct((m, n4), jnp.float32),
        compiler_params=pltpu.CompilerParams(
            dimension_semantics=("arbitrary", "arbitrary")),
    )(x, ids3, w, bias)


def _encoder(xw_seq, w_hh):
    s_len, b, h4 = xw_seq.shape
    h = h4 // 4

    def body(xw_ref, whh_ref, h_out, c_out, h_scr, c_scr, whh_bf):
        s = pl.program_id(0)

        @pl.when(s == 0)
        def _():
            h_scr[...] = jnp.zeros_like(h_scr)
            c_scr[...] = jnp.zeros_like(c_scr)
            whh_bf[...] = whh_ref[...].astype(jnp.bfloat16)

        hprev = h_scr[...]
        c = c_scr[...]
        gates = xw_ref[0] + _dot_t(hprev.astype(jnp.bfloat16), whh_bf[...])
        i, f, g, o = _split_gates(gates, h)
        c2 = f * c + i * g
        h2 = o * jnp.tanh(c2)
        h_scr[...] = h2
        c_scr[...] = c2

        @pl.when(s == s_len - 1)
        def _():
            h_out[...] = h2
            c_out[...] = c2

    return pl.pallas_call(
        body,
        grid=(s_len,),
        in_specs=[
            pl.BlockSpec((1, b, h4), lambda s: (s, 0, 0)),
            pl.BlockSpec(w_hh.shape, lambda s: (0, 0)),
        ],
        out_specs=[
            pl.BlockSpec((b, h), lambda s: (0, 0)),
            pl.BlockSpec((b, h), lambda s: (0, 0)),
        ],
        out_shape=[
            jax.ShapeDtypeStruct((b, h), jnp.float32),
            jax.ShapeDtypeStruct((b, h), jnp.float32),
        ],
        scratch_shapes=[
            pltpu.VMEM((b, h), jnp.float32),
            pltpu.VMEM((b, h), jnp.float32),
            pltpu.VMEM(w_hh.shape, jnp.bfloat16),
        ],
        compiler_params=pltpu.CompilerParams(
            dimension_semantics=("arbitrary",)),
    )(xw_seq, w_hh)


def _decoder(xw_seq, w_hh, h_enc, c_enc, w_tl_h, w_tl_e, b_tl):
    s_len, b, h4 = xw_seq.shape
    h = h4 // 4

    def body(xw_ref, whh_ref, he_ref, ce_ref, wtlh_ref, wtle_ref, btl_ref,
             comb_out, h_scr, c_scr, const_scr, whh_bf, wtlh_bf):
        s = pl.program_id(0)

        @pl.when(s == 0)
        def _():
            h_scr[...] = he_ref[...]
            c_scr[...] = ce_ref[...]
            const_scr[...] = _dot_t(he_ref[...], wtle_ref[...]) + btl_ref[...]
            whh_bf[...] = whh_ref[...].astype(jnp.bfloat16)
            wtlh_bf[...] = wtlh_ref[...].astype(jnp.bfloat16)

        hprev = h_scr[...]
        c = c_scr[...]
        gates = xw_ref[0] + _dot_t(hprev.astype(jnp.bfloat16), whh_bf[...])
        i, f, g, o = _split_gates(gates, h)
        c2 = f * c + i * g
        h2 = o * jnp.tanh(c2)
        h_scr[...] = h2
        c_scr[...] = c2
        comb_out[0] = jnp.tanh(
            _dot_t(h2.astype(jnp.bfloat16), wtlh_bf[...]) + const_scr[...])

    return pl.pallas_call(
        body,
        grid=(s_len,),
        in_specs=[
            pl.BlockSpec((1, b, h4), lambda s: (s, 0, 0)),
            pl.BlockSpec(w_hh.shape, lambda s: (0, 0)),
            pl.BlockSpec((b, h), lambda s: (0, 0)),
            pl.BlockSpec((b, h), lambda s: (0, 0)),
            pl.BlockSpec(w_tl_h.shape, lambda s: (0, 0)),
            pl.BlockSpec(w_tl_e.shape, lambda s: (0, 0)),
            pl.BlockSpec(b_tl.shape, lambda s: (0, 0)),
        ],
        out_specs=pl.BlockSpec((1, b, h), lambda s: (s, 0, 0)),
        out_shape=jax.ShapeDtypeStruct((s_len, b, h), jnp.float32),
        scratch_shapes=[
            pltpu.VMEM((b, h), jnp.float32),
            pltpu.VMEM((b, h), jnp.float32),
            pltpu.VMEM((b, h), jnp.float32),
            pltpu.VMEM(w_hh.shape, jnp.bfloat16),
            pltpu.VMEM(w_tl_h.shape, jnp.bfloat16),
        ],
        compiler_params=pltpu.CompilerParams(
            dimension_semantics=("arbitrary",)),
    )(xw_seq, w_hh, h_enc, c_enc, w_tl_h, w_tl_e, b_tl)


def _project(comb, w_lin, b_lin):
    m, h = comb.shape
    v = w_lin.shape[0]
    bn = 1024

    def body(c_ref, w_ref, b_ref, o_ref):
        o_ref[...] = _dot_t(c_ref[...], w_ref[...]) + b_ref[...]

    return pl.pallas_call(
        body,
        grid=(v // bn,),
        in_specs=[
            pl.BlockSpec((m, h), lambda n: (0, 0)),
            pl.BlockSpec((bn, h), lambda n: (n, 0)),
            pl.BlockSpec((1, bn), lambda n: (0, n)),
        ],
        out_specs=pl.BlockSpec((m, bn), lambda n: (0, n)),
        out_shape=jax.ShapeDtypeStruct((m, v), jnp.float32),
        compiler_params=pltpu.CompilerParams(
            dimension_semantics=("arbitrary",)),
    )(comb, w_lin, b_lin)


def kernel(input_ids, target_ids, emb_in, emb_tgt, W_ih_e, W_hh_e, b_ih_e,
           b_hh_e, W_ih_d, W_hh_d, b_ih_d, b_hh_d, W_tl, b_tl, W_lin, b_lin):
    b, s_in = input_ids.shape
    s_out = target_ids.shape[1]
    h = W_hh_e.shape[1]
    v = W_lin.shape[0]

    ids_in = input_ids.T.reshape(-1)
    ids_tgt = target_ids.T.reshape(-1)
    x_in_flat, x_tgt_flat = _sc_gather_pair(emb_in, ids_in, emb_tgt, ids_tgt)

    bm = 512
    xw_in = _masked_xw(x_in_flat, ids_in.reshape(s_in * b // bm, 1, bm),
                       W_ih_e, (b_ih_e + b_hh_e).reshape(1, -1))
    xw_tgt = _masked_xw(x_tgt_flat, ids_tgt.reshape(s_out * b // bm, 1, bm),
                        W_ih_d, (b_ih_d + b_hh_d).reshape(1, -1))
    h_enc, c_enc = _encoder(xw_in.reshape(s_in, b, 4 * h), W_hh_e)
    comb = _decoder(xw_tgt.reshape(s_out, b, 4 * h), W_hh_d,
                    h_enc, c_enc, W_tl[:, :h], W_tl[:, h:],
                    b_tl.reshape(1, -1))
    comb_flat = comb.transpose(1, 0, 2).reshape(b * s_out, h)
    logits = _project(comb_flat, W_lin, b_lin.reshape(1, -1))
    return logits.reshape(b, s_out, v)


# speedup vs baseline: 1.0043x; 1.0043x over previous
import functools

import jax
import jax.numpy as jnp
from jax import lax
from jax.experimental import pallas as pl
from jax.experimental.pallas import tpu as pltpu
from jax.experimental.pallas import tpu_sc as plsc


def _dot_t(a, w):
    return lax.dot_general(a, w, (((1,), (1,)), ((), ())),
                           preferred_element_type=jnp.float32)


def _sc_gather_pair(emb_a, idx_a, emb_b, idx_b):
    n = idx_a.shape[0]
    h = emb_a.shape[1]
    info = plsc.get_sparse_core_info()
    nw = info.num_cores * info.num_subcores
    n_per = n // nw

    nc = n_per // 2

    @functools.partial(
        pl.kernel,
        out_type=(jax.ShapeDtypeStruct((n, h), jnp.float32),
                  jax.ShapeDtypeStruct((n, h), jnp.float32)),
        mesh=plsc.VectorSubcoreMesh(core_axis_name="c", subcore_axis_name="s"),
        scratch_types=[
            pltpu.VMEM((nc,), jnp.int32),
            pltpu.VMEM((nc,), jnp.int32),
            pltpu.VMEM((nc,), jnp.int32),
            pltpu.VMEM((nc,), jnp.int32),
            pltpu.VMEM((nc, h), jnp.float32),
            pltpu.VMEM((nc, h), jnp.float32),
            pltpu.SemaphoreType.DMA,
            pltpu.SemaphoreType.DMA,
        ],
    )
    def k(emb_a_hbm, idx_a_hbm, emb_b_hbm, idx_b_hbm, out_a, out_b,
          ia0, ia1, ib0, ib1, r0, r1, s0, s1):
        wid = lax.axis_index("s") * info.num_cores + lax.axis_index("c")
        base = wid * n_per
        pltpu.sync_copy(idx_a_hbm.at[pl.ds(base, nc)], ia0)
        pltpu.sync_copy(idx_a_hbm.at[pl.ds(base + nc, nc)], ia1)
        pltpu.sync_copy(idx_b_hbm.at[pl.ds(base, nc)], ib0)
        pltpu.sync_copy(idx_b_hbm.at[pl.ds(base + nc, nc)], ib1)
        c0 = pltpu.async_copy(emb_a_hbm.at[ia0], r0, s0)
        c1 = pltpu.async_copy(emb_a_hbm.at[ia1], r1, s1)
        c0.wait()
        pltpu.sync_copy(r0, out_a.at[pl.ds(base, nc)])
        c2 = pltpu.async_copy(emb_b_hbm.at[ib0], r0, s0)
        c1.wait()
        pltpu.sync_copy(r1, out_a.at[pl.ds(base + nc, nc)])
        c3 = pltpu.async_copy(emb_b_hbm.at[ib1], r1, s1)
        c2.wait()
        pltpu.sync_copy(r0, out_b.at[pl.ds(base, nc)])
        c3.wait()
        pltpu.sync_copy(r1, out_b.at[pl.ds(base + nc, nc)])

    return k(emb_a, idx_a, emb_b, idx_b)


def _split_gates(gates, hh):
    i = jax.nn.sigmoid(gates[:, :hh])
    f = jax.nn.sigmoid(gates[:, hh:2 * hh])
    g = jnp.tanh(gates[:, 2 * hh:3 * hh])
    o = jax.nn.sigmoid(gates[:, 3 * hh:])
    return i, f, g, o


def _masked_xw(x, ids3, w, bias):
    m, h = x.shape
    n4 = w.shape[0]
    bm, bn = 512, 2048

    def body(x_ref, ids_ref, w_ref, b_ref, o_ref):
        mask = (ids_ref[0, 0, :] != 0).astype(jnp.float32)
        o_ref[...] = _dot_t(x_ref[...] * mask[:, None], w_ref[...]) + b_ref[...]

    return pl.pallas_call(
        body,
        grid=(n4 // bn, m // bm),
        in_specs=[
            pl.BlockSpec((bm, h), lambda n, mm: (mm, 0)),
            pl.BlockSpec((1, 1, bm), lambda n, mm: (mm, 0, 0)),
            pl.BlockSpec((bn, h), lambda n, mm: (n, 0)),
            pl.BlockSpec((1, bn), lambda n, mm: (0, n)),
        ],
        out_specs=pl.BlockSpec((bm, bn), lambda n, mm: (mm, n)),
        out_shape=jax.ShapeDtypeStruct((m, n4), jnp.float32),
        compiler_params=pltpu.CompilerParams(
            dimension_semantics=("arbitrary", "arbitrary")),
    )(x, ids3, w, bias)


def _encoder(xw_seq, w_hh):
    s_len, b, h4 = xw_seq.shape
    h = h4 // 4

    def body(xw_ref, whh_ref, h_out, c_out, h_scr, c_scr):
        s = pl.program_id(0)

        @pl.when(s == 0)
        def _():
            h_scr[...] = jnp.zeros_like(h_scr)
            c_scr[...] = jnp.zeros_like(c_scr)

        hprev = h_scr[...]
        c = c_scr[...]
        gates = xw_ref[0] + _dot_t(hprev, whh_ref[...])
        i, f, g, o = _split_gates(gates, h)
        c2 = f * c + i * g
        h2 = o * jnp.tanh(c2)
        h_scr[...] = h2
        c_scr[...] = c2

        @pl.when(s == s_len - 1)
        def _():
            h_out[...] = h2
            c_out[...] = c2

    return pl.pallas_call(
        body,
        grid=(s_len,),
        in_specs=[
            pl.BlockSpec((1, b, h4), lambda s: (s, 0, 0)),
            pl.BlockSpec(w_hh.shape, lambda s: (0, 0)),
        ],
        out_specs=[
            pl.BlockSpec((b, h), lambda s: (0, 0)),
            pl.BlockSpec((b, h), lambda s: (0, 0)),
        ],
        out_shape=[
            jax.ShapeDtypeStruct((b, h), jnp.float32),
            jax.ShapeDtypeStruct((b, h), jnp.float32),
        ],
        scratch_shapes=[
            pltpu.VMEM((b, h), jnp.float32),
            pltpu.VMEM((b, h), jnp.float32),
        ],
        compiler_params=pltpu.CompilerParams(
            dimension_semantics=("arbitrary",)),
    )(xw_seq, w_hh)


def _decoder(xw_seq, w_hh, h_enc, c_enc, w_tl_h, w_tl_e, b_tl):
    s_len, b, h4 = xw_seq.shape
    h = h4 // 4

    def body(xw_ref, whh_ref, he_ref, ce_ref, wtlh_ref, wtle_ref, btl_ref,
             comb_out, h_scr, c_scr, const_scr):
        s = pl.program_id(0)

        @pl.when(s == 0)
        def _():
            h_scr[...] = he_ref[...]
            c_scr[...] = ce_ref[...]
            const_scr[...] = _dot_t(he_ref[...], wtle_ref[...]) + btl_ref[...]

        hprev = h_scr[...]
        c = c_scr[...]
        gates = xw_ref[0] + _dot_t(hprev, whh_ref[...])
        i, f, g, o = _split_gates(gates, h)
        c2 = f * c + i * g
        h2 = o * jnp.tanh(c2)
        h_scr[...] = h2
        c_scr[...] = c2
        comb_out[:, 0, 0, :] = jnp.tanh(
            _dot_t(h2, wtlh_ref[...]) + const_scr[...])

    return pl.pallas_call(
        body,
        grid=(s_len,),
        in_specs=[
            pl.BlockSpec((1, b, h4), lambda s: (s, 0, 0)),
            pl.BlockSpec(w_hh.shape, lambda s: (0, 0)),
            pl.BlockSpec((b, h), lambda s: (0, 0)),
            pl.BlockSpec((b, h), lambda s: (0, 0)),
            pl.BlockSpec(w_tl_h.shape, lambda s: (0, 0)),
            pl.BlockSpec(w_tl_e.shape, lambda s: (0, 0)),
            pl.BlockSpec(b_tl.shape, lambda s: (0, 0)),
        ],
        out_specs=pl.BlockSpec((b, 1, 1, h), lambda s: (0, s, 0, 0)),
        out_shape=jax.ShapeDtypeStruct((b, s_len, 1, h), jnp.float32),
        scratch_shapes=[
            pltpu.VMEM((b, h), jnp.float32),
            pltpu.VMEM((b, h), jnp.float32),
            pltpu.VMEM((b, h), jnp.float32),
        ],
        compiler_params=pltpu.CompilerParams(
            dimension_semantics=("arbitrary",)),
    )(xw_seq, w_hh, h_enc, c_enc, w_tl_h, w_tl_e, b_tl)


def _project(comb, w_lin, b_lin):
    m, h = comb.shape
    v = w_lin.shape[0]
    bn = 1024

    def body(c_ref, w_ref, b_ref, o_ref):
        o_ref[...] = _dot_t(c_ref[...], w_ref[...]) + b_ref[...]

    return pl.pallas_call(
        body,
        grid=(v // bn,),
        in_specs=[
            pl.BlockSpec((m, h), lambda n: (0, 0)),
            pl.BlockSpec((bn, h), lambda n: (n, 0)),
            pl.BlockSpec((1, bn), lambda n: (0, n)),
        ],
        out_specs=pl.BlockSpec((m, bn), lambda n: (0, n)),
        out_shape=jax.ShapeDtypeStruct((m, v), jnp.float32),
        compiler_params=pltpu.CompilerParams(
            dimension_semantics=("arbitrary",)),
    )(comb, w_lin, b_lin)


def kernel(input_ids, target_ids, emb_in, emb_tgt, W_ih_e, W_hh_e, b_ih_e,
           b_hh_e, W_ih_d, W_hh_d, b_ih_d, b_hh_d, W_tl, b_tl, W_lin, b_lin):
    b, s_in = input_ids.shape
    s_out = target_ids.shape[1]
    h = W_hh_e.shape[1]
    v = W_lin.shape[0]

    ids_in = input_ids.T.reshape(-1)
    ids_tgt = target_ids.T.reshape(-1)
    x_in_flat, x_tgt_flat = _sc_gather_pair(emb_in, ids_in, emb_tgt, ids_tgt)

    bm = 512
    xw_in = _masked_xw(x_in_flat, ids_in.reshape(s_in * b // bm, 1, bm),
                       W_ih_e, (b_ih_e + b_hh_e).reshape(1, -1))
    xw_tgt = _masked_xw(x_tgt_flat, ids_tgt.reshape(s_out * b // bm, 1, bm),
                        W_ih_d, (b_ih_d + b_hh_d).reshape(1, -1))
    h_enc, c_enc = _encoder(xw_in.reshape(s_in, b, 4 * h), W_hh_e)
    comb = _decoder(xw_tgt.reshape(s_out, b, 4 * h), W_hh_d,
                    h_enc, c_enc, W_tl[:, :h], W_tl[:, h:],
                    b_tl.reshape(1, -1))
    comb_flat = comb.reshape(b * s_out, h)
    logits = _project(comb_flat, W_lin, b_lin.reshape(1, -1))
    return logits.reshape(b, s_out, v)


# speedup vs baseline: 1.0063x; 1.0020x over previous
import functools

import jax
import jax.numpy as jnp
from jax import lax
from jax.experimental import pallas as pl
from jax.experimental.pallas import tpu as pltpu
from jax.experimental.pallas import tpu_sc as plsc


def _dot_t(a, w):
    return lax.dot_general(a, w, (((1,), (1,)), ((), ())),
                           preferred_element_type=jnp.float32)


def _sc_gather(emb, idx):
    n = idx.shape[0]
    h = emb.shape[1]
    info = plsc.get_sparse_core_info()
    nw = info.num_cores * info.num_subcores
    n_per = n // nw
    nc = n_per // 2

    @functools.partial(
        pl.kernel,
        out_type=jax.ShapeDtypeStruct((n, h), jnp.float32),
        mesh=plsc.VectorSubcoreMesh(core_axis_name="c", subcore_axis_name="s"),
        scratch_types=[
            pltpu.VMEM((nc,), jnp.int32),
            pltpu.VMEM((nc,), jnp.int32),
            pltpu.VMEM((nc, h), jnp.float32),
            pltpu.VMEM((nc, h), jnp.float32),
            pltpu.SemaphoreType.DMA,
            pltpu.SemaphoreType.DMA,
        ],
    )
    def k(emb_hbm, idx_hbm, out, i0, i1, r0, r1, s0, s1):
        wid = lax.axis_index("s") * info.num_cores + lax.axis_index("c")
        base = wid * n_per
        pltpu.sync_copy(idx_hbm.at[pl.ds(base, nc)], i0)
        pltpu.sync_copy(idx_hbm.at[pl.ds(base + nc, nc)], i1)
        c0 = pltpu.async_copy(emb_hbm.at[i0], r0, s0)
        c1 = pltpu.async_copy(emb_hbm.at[i1], r1, s1)
        c0.wait()
        pltpu.sync_copy(r0, out.at[pl.ds(base, nc)])
        c1.wait()
        pltpu.sync_copy(r1, out.at[pl.ds(base + nc, nc)])

    return k(emb, idx)


def _split_gates(gates, hh):
    i = jax.nn.sigmoid(gates[:, :hh])
    f = jax.nn.sigmoid(gates[:, hh:2 * hh])
    g = jnp.tanh(gates[:, 2 * hh:3 * hh])
    o = jax.nn.sigmoid(gates[:, 3 * hh:])
    return i, f, g, o


def _masked_xw(x, ids3, w, bias):
    m, h = x.shape
    n4 = w.shape[0]
    bm, bn = 512, 2048

    def body(x_ref, ids_ref, w_ref, b_ref, o_ref):
        mask = (ids_ref[0, 0, :] != 0).astype(jnp.float32)
        o_ref[...] = _dot_t(x_ref[...] * mask[:, None], w_ref[...]) + b_ref[...]

    return pl.pallas_call(
        body,
        grid=(n4 // bn, m // bm),
        in_specs=[
            pl.BlockSpec((bm, h), lambda n, mm: (mm, 0)),
            pl.BlockSpec((1, 1, bm), lambda n, mm: (mm, 0, 0)),
            pl.BlockSpec((bn, h), lambda n, mm: (n, 0)),
            pl.BlockSpec((1, bn), lambda n, mm: (0, n)),
        ],
        out_specs=pl.BlockSpec((bm, bn), lambda n, mm: (mm, n)),
        out_shape=jax.ShapeDtypeStruct((m, n4), jnp.float32),
        compiler_params=pltpu.CompilerParams(
            dimension_semantics=("arbitrary", "arbitrary")),
    )(x, ids3, w, bias)


def _encoder(xw_seq, w_hh):
    s_len, b, h4 = xw_seq.shape
    h = h4 // 4

    def body(xw_ref, whh_ref, h_out, c_out, h_scr, c_scr):
        s = pl.program_id(0)

        @pl.when(s == 0)
        def _():
            h_scr[...] = jnp.zeros_like(h_scr)
            c_scr[...] = jnp.zeros_like(c_scr)

        hprev = h_scr[...]
        c = c_scr[...]
        gates = xw_ref[0] + _dot_t(hprev, whh_ref[...])
        i, f, g, o = _split_gates(gates, h)
        c2 = f * c + i * g
        h2 = o * jnp.tanh(c2)
        h_scr[...] = h2
        c_scr[...] = c2

        @pl.when(s == s_len - 1)
        def _():
            h_out[...] = h2
            c_out[...] = c2

    return pl.pallas_call(
        body,
        grid=(s_len,),
        in_specs=[
            pl.BlockSpec((1, b, h4), lambda s: (s, 0, 0)),
            pl.BlockSpec(w_hh.shape, lambda s: (0, 0)),
        ],
        out_specs=[
            pl.BlockSpec((b, h), lambda s: (0, 0)),
            pl.BlockSpec((b, h), lambda s: (0, 0)),
        ],
        out_shape=[
            jax.ShapeDtypeStruct((b, h), jnp.float32),
            jax.ShapeDtypeStruct((b, h), jnp.float32),
        ],
        scratch_shapes=[
            pltpu.VMEM((b, h), jnp.float32),
            pltpu.VMEM((b, h), jnp.float32),
        ],
        compiler_params=pltpu.CompilerParams(
            dimension_semantics=("arbitrary",)),
    )(xw_seq, w_hh)


def _decoder(xw_seq, w_hh, h_enc, c_enc, w_tl_h, w_tl_e, b_tl):
    s_len, b, h4 = xw_seq.shape
    h = h4 // 4

    def body(xw_ref, whh_ref, he_ref, ce_ref, wtlh_ref, wtle_ref, btl_ref,
             comb_out, h_scr, c_scr, const_scr):
        s = pl.program_id(0)

        @pl.when(s == 0)
        def _():
            h_scr[...] = he_ref[...]
            c_scr[...] = ce_ref[...]
            const_scr[...] = _dot_t(he_ref[...], wtle_ref[...]) + btl_ref[...]

        hprev = h_scr[...]
        c = c_scr[...]
        gates = xw_ref[0] + _dot_t(hprev, whh_ref[...])
        i, f, g, o = _split_gates(gates, h)
        c2 = f * c + i * g
        h2 = o * jnp.tanh(c2)
        h_scr[...] = h2
        c_scr[...] = c2
        comb_out[:, 0, 0, :] = jnp.tanh(
            _dot_t(h2, wtlh_ref[...]) + const_scr[...])

    return pl.pallas_call(
        body,
        grid=(s_len,),
        in_specs=[
            pl.BlockSpec((1, b, h4), lambda s: (s, 0, 0)),
            pl.BlockSpec(w_hh.shape, lambda s: (0, 0)),
            pl.BlockSpec((b, h), lambda s: (0, 0)),
            pl.BlockSpec((b, h), lambda s: (0, 0)),
            pl.BlockSpec(w_tl_h.shape, lambda s: (0, 0)),
            pl.BlockSpec(w_tl_e.shape, lambda s: (0, 0)),
            pl.BlockSpec(b_tl.shape, lambda s: (0, 0)),
        ],
        out_specs=pl.BlockSpec((b, 1, 1, h), lambda s: (0, s, 0, 0)),
        out_shape=jax.ShapeDtypeStruct((b, s_len, 1, h), jnp.float32),
        scratch_shapes=[
            pltpu.VMEM((b, h), jnp.float32),
            pltpu.VMEM((b, h), jnp.float32),
            pltpu.VMEM((b, h), jnp.float32),
        ],
        compiler_params=pltpu.CompilerParams(
            dimension_semantics=("arbitrary",)),
    )(xw_seq, w_hh, h_enc, c_enc, w_tl_h, w_tl_e, b_tl)


def _project(comb, w_lin, b_lin):
    m, h = comb.shape
    v = w_lin.shape[0]
    bn = 1024

    def body(c_ref, w_ref, b_ref, o_ref):
        o_ref[...] = _dot_t(c_ref[...], w_ref[...]) + b_ref[...]

    return pl.pallas_call(
        body,
        grid=(v // bn,),
        in_specs=[
            pl.BlockSpec((m, h), lambda n: (0, 0)),
            pl.BlockSpec((bn, h), lambda n: (n, 0)),
            pl.BlockSpec((1, bn), lambda n: (0, n)),
        ],
        out_specs=pl.BlockSpec((m, bn), lambda n: (0, n)),
        out_shape=jax.ShapeDtypeStruct((m, v), jnp.float32),
        compiler_params=pltpu.CompilerParams(
            dimension_semantics=("arbitrary",)),
    )(comb, w_lin, b_lin)


def kernel(input_ids, target_ids, emb_in, emb_tgt, W_ih_e, W_hh_e, b_ih_e,
           b_hh_e, W_ih_d, W_hh_d, b_ih_d, b_hh_d, W_tl, b_tl, W_lin, b_lin):
    b, s_in = input_ids.shape
    s_out = target_ids.shape[1]
    h = W_hh_e.shape[1]
    v = W_lin.shape[0]

    ids_in = input_ids.T.reshape(-1)
    ids_tgt = target_ids.T.reshape(-1)
    x_in_flat = _sc_gather(emb_in, ids_in)
    x_tgt_flat = _sc_gather(emb_tgt, ids_tgt)

    bm = 512
    xw_in = _masked_xw(x_in_flat, ids_in.reshape(s_in * b // bm, 1, bm),
                       W_ih_e, (b_ih_e + b_hh_e).reshape(1, -1))
    xw_tgt = _masked_xw(x_tgt_flat, ids_tgt.reshape(s_out * b // bm, 1, bm),
                        W_ih_d, (b_ih_d + b_hh_d).reshape(1, -1))
    h_enc, c_enc = _encoder(xw_in.reshape(s_in, b, 4 * h), W_hh_e)
    comb = _decoder(xw_tgt.reshape(s_out, b, 4 * h), W_hh_d,
                    h_enc, c_enc, W_tl[:, :h], W_tl[:, h:],
                    b_tl.reshape(1, -1))
    comb_flat = comb.reshape(b * s_out, h)
    logits = _project(comb_flat, W_lin, b_lin.reshape(1, -1))
    return logits.reshape(b, s_out, v)
